# baseline (speedup 1.0000x reference)
import functools

import jax
import jax.numpy as jnp
from jax import lax
from jax.experimental import pallas as pl
from jax.experimental.pallas import tpu as pltpu
from jax.experimental.pallas import tpu_sc as plsc

VOCAB = 1000
VPAD = 1024
EMBED = 64
B = 1024
S = 50

NC = 2
NS = 16
NW = NC * NS
NTOK = B * S
BPW = NTOK // NW
CHUNK = 64
NCHUNK = BPW // CHUNK


def _matmul_body(e_ref, w_ref, t_ref):
    t_ref[...] = lax.dot_general(
        e_ref[...], w_ref[...],
        dimension_numbers=(((1,), (1,)), ((), ())),
        preferred_element_type=jnp.float32,
    )


def _compute_table(embed_table, head_weight):
    return pl.pallas_call(
        _matmul_body,
        out_shape=jax.ShapeDtypeStruct((VOCAB, VOCAB), jnp.float32),
    )(embed_table, head_weight)


def _gather_body(table_hbm, idx_hbm, out_hbm, idx_v, rows_v, sem):
    wid = lax.axis_index("s") * NC + lax.axis_index("c")
    base = wid * BPW
    pltpu.sync_copy(idx_hbm.at[pl.ds(base, BPW)], idx_v)

    def body(c, carry):
        off = c * CHUNK
        pltpu.async_copy(
            table_hbm.at[idx_v.at[pl.ds(off, CHUNK)]], rows_v, sem
        ).wait()
        pltpu.sync_copy(rows_v, out_hbm.at[pl.ds(base + off, CHUNK)])
        return carry

    lax.fori_loop(0, NCHUNK, body, 0)


_gather_call = functools.partial(
    pl.kernel,
    out_type=jax.ShapeDtypeStruct((NTOK, VOCAB), jnp.float32),
    mesh=plsc.VectorSubcoreMesh(core_axis_name="c", subcore_axis_name="s"),
    scratch_types=[
        pltpu.VMEM((BPW,), jnp.int32),
        pltpu.VMEM((CHUNK, VOCAB), jnp.float32),
        pltpu.SemaphoreType.DMA,
    ],
    compiler_params=pltpu.CompilerParams(use_tc_tiling_on_sc=False),
)(_gather_body)


def kernel(x, embed_table, head_weight):
    table = _compute_table(embed_table, head_weight)
    logits = _gather_call(table, x.reshape(NTOK))
    return logits.reshape(B, S, VOCAB)


# speedup vs baseline: 1.0033x; 1.0033x over previous
import functools

import jax
import jax.numpy as jnp
from jax import lax
from jax.experimental import pallas as pl
from jax.experimental.pallas import tpu as pltpu
from jax.experimental.pallas import tpu_sc as plsc

VOCAB = 1000
VPAD = 1024
EMBED = 64
B = 1024
S = 50

NC = 2
NS = 16
NW = NC * NS
NTOK = B * S
BPW = NTOK // NW
CHUNK = 64
NCHUNK = BPW // CHUNK


def _matmul_body(e_ref, w_ref, t_ref):
    t_ref[...] = lax.dot_general(
        e_ref[...], w_ref[...],
        dimension_numbers=(((1,), (1,)), ((), ())),
        preferred_element_type=jnp.float32,
    )


def _compute_table(embed_table, head_weight):
    return pl.pallas_call(
        _matmul_body,
        out_shape=jax.ShapeDtypeStruct((VOCAB, VOCAB), jnp.float32),
    )(embed_table, head_weight)


def _gather_body(table_hbm, idx_hbm, out_hbm, idx_v, rows0, rows1,
                 gsem0, gsem1, wsem0, wsem1):
    wid = lax.axis_index("s") * NC + lax.axis_index("c")
    base = wid * BPW
    pltpu.sync_copy(idx_hbm.at[pl.ds(base, BPW)], idx_v)

    rows = (rows0, rows1)
    gsem = (gsem0, gsem1)
    wsem = (wsem0, wsem1)

    def start_gather(c):
        b = c % 2
        return pltpu.async_copy(
            table_hbm.at[idx_v.at[pl.ds(c * CHUNK, CHUNK)]], rows[b], gsem[b]
        )

    def start_write(c):
        b = c % 2
        return pltpu.async_copy(
            rows[b], out_hbm.at[pl.ds(base + c * CHUNK, CHUNK)], wsem[b]
        )

    gathers = [None] * NCHUNK
    writes = [None] * NCHUNK
    gathers[0] = start_gather(0)
    for c in range(NCHUNK):
        gathers[c].wait()
        if c + 1 < NCHUNK:
            if c >= 1:
                writes[c - 1].wait()
            gathers[c + 1] = start_gather(c + 1)
        writes[c] = start_write(c)
    writes[NCHUNK - 1].wait()


_gather_call = functools.partial(
    pl.kernel,
    out_type=jax.ShapeDtypeStruct((NTOK, VOCAB), jnp.float32),
    mesh=plsc.VectorSubcoreMesh(core_axis_name="c", subcore_axis_name="s"),
    scratch_types=[
        pltpu.VMEM((BPW,), jnp.int32),
        pltpu.VMEM((CHUNK, VOCAB), jnp.float32),
        pltpu.VMEM((CHUNK, VOCAB), jnp.float32),
        pltpu.SemaphoreType.DMA,
        pltpu.SemaphoreType.DMA,
        pltpu.SemaphoreType.DMA,
        pltpu.SemaphoreType.DMA,
    ],
    compiler_params=pltpu.CompilerParams(use_tc_tiling_on_sc=False),
)(_gather_body)


def kernel(x, embed_table, head_weight):
    table = _compute_table(embed_table, head_weight)
    logits = _gather_call(table, x.reshape(NTOK))
    return logits.reshape(B, S, VOCAB)


# speedup vs baseline: 1.0050x; 1.0017x over previous
import functools

import jax
import jax.numpy as jnp
from jax import lax
from jax.experimental import pallas as pl
from jax.experimental.pallas import tpu as pltpu
from jax.experimental.pallas import tpu_sc as plsc

VOCAB = 1000
VPAD = 1024
EMBED = 64
B = 1024
S = 50

NC = 2
NS = 16
NW = NC * NS
NTOK = B * S
BPW = NTOK // NW
CHUNK = 64
NCHUNK = BPW // CHUNK


def _matmul_body(e_ref, w_ref, t_ref):
    t_ref[...] = lax.dot_general(
        e_ref[...], w_ref[...],
        dimension_numbers=(((1,), (1,)), ((), ())),
        preferred_element_type=jnp.float32,
    )


def _compute_table(embed_table, head_weight):
    return pl.pallas_call(
        _matmul_body,
        out_shape=jax.ShapeDtypeStruct((VOCAB, VOCAB), jnp.float32),
    )(embed_table, head_weight)


BPW_B = B // NW


def _gather_body(table_hbm, idx_hbm, out_hbm, idx_v, rows0, rows1,
                 gsem0, gsem1, wsem0, wsem1):
    wid = lax.axis_index("s") * NC + lax.axis_index("c")
    b0 = wid * BPW_B
    pltpu.sync_copy(idx_hbm.at[pl.ds(b0, BPW_B)], idx_v)

    rows = (rows0, rows1)
    gsem = (gsem0, gsem1)
    wsem = (wsem0, wsem1)

    def start_gather(c):
        b = c % 2
        return pltpu.async_copy(
            table_hbm.at[idx_v.at[c]], rows[b], gsem[b]
        )

    def start_write(c):
        b = c % 2
        return pltpu.async_copy(rows[b], out_hbm.at[b0 + c], wsem[b])

    gathers = [None] * BPW_B
    writes = [None] * BPW_B
    gathers[0] = start_gather(0)
    for c in range(BPW_B):
        gathers[c].wait()
        if c + 1 < BPW_B:
            if c >= 1:
                writes[c - 1].wait()
            gathers[c + 1] = start_gather(c + 1)
        writes[c] = start_write(c)
    writes[BPW_B - 1].wait()


_gather_call = functools.partial(
    pl.kernel,
    out_type=jax.ShapeDtypeStruct((B, S, VOCAB), jnp.float32),
    mesh=plsc.VectorSubcoreMesh(core_axis_name="c", subcore_axis_name="s"),
    scratch_types=[
        pltpu.VMEM((BPW_B, S), jnp.int32),
        pltpu.VMEM((S, VOCAB), jnp.float32),
        pltpu.VMEM((S, VOCAB), jnp.float32),
        pltpu.SemaphoreType.DMA,
        pltpu.SemaphoreType.DMA,
        pltpu.SemaphoreType.DMA,
        pltpu.SemaphoreType.DMA,
    ],
    compiler_params=pltpu.CompilerParams(use_tc_tiling_on_sc=False),
)(_gather_body)


def kernel(x, embed_table, head_weight):
    table = _compute_table(embed_table, head_weight)
    return _gather_call(table, x)


# speedup vs baseline: 1.5047x; 1.4972x over previous
import functools

import jax
import jax.numpy as jnp
from jax import lax
from jax.experimental import pallas as pl
from jax.experimental.pallas import tpu as pltpu
from jax.experimental.pallas import tpu_sc as plsc

VOCAB = 1000
VPAD = 1024
NBLK = VPAD // 128
EMBED = 64
B = 1024
S = 50
SPAD = 56

NC = 2
NS = 16
NW = NC * NS
BPW = B // NW
IDX_PER_W = NBLK * BPW * SPAD


def _matmul_body(e_ref, w_ref, t_ref):
    t_ref[...] = lax.dot_general(
        e_ref[...], w_ref[...],
        dimension_numbers=(((1,), (1,)), ((), ())),
        preferred_element_type=jnp.float32,
    )


def _compute_table(embed_table, head_weight):
    w_pad = jnp.zeros((VPAD, EMBED), jnp.float32).at[:VOCAB].set(head_weight)
    return pl.pallas_call(
        _matmul_body,
        out_shape=jax.ShapeDtypeStruct((VOCAB, VPAD), jnp.float32),
    )(embed_table, w_pad)


def _gather_body(tt_hbm, idx_hbm, out_hbm, idx_v, rows, tail,
                 gsem, tsem, wsem):
    wid = lax.axis_index("s") * NC + lax.axis_index("c")
    pltpu.sync_copy(idx_hbm.at[pl.ds(wid * IDX_PER_W, IDX_PER_W)], idx_v)
    b0 = wid * BPW

    def batch_body(c, carry):
        coff = pl.multiple_of(SPAD * c, 8)
        gathers = []
        for tv in range(NBLK - 1):
            gathers.append(pltpu.async_copy(
                tt_hbm.at[idx_v.at[pl.ds(tv * (BPW * SPAD) + coff, S)]],
                rows.at[:, pl.ds(128 * tv, 128)],
                gsem,
            ))
        tg = pltpu.async_copy(
            tt_hbm.at[idx_v.at[pl.ds((NBLK - 1) * (BPW * SPAD) + coff, S)]],
            tail, tsem)
        for g in gathers:
            g.wait()
        tg.wait()

        def tail_body(s, inner):
            rows[s, pl.ds(984, 16)] = tail[s, pl.ds(88, 16)]
            for j in range(6):
                rows[s, pl.ds(896 + 16 * j, 16)] = tail[s, pl.ds(16 * j, 16)]
            return inner

        lax.fori_loop(0, S, tail_body, 0)
        pltpu.async_copy(rows, out_hbm.at[b0 + c], wsem).wait()
        return carry

    lax.fori_loop(0, BPW, batch_body, 0)


_gather_call = functools.partial(
    pl.kernel,
    out_type=jax.ShapeDtypeStruct((B, S, VOCAB), jnp.float32),
    mesh=plsc.VectorSubcoreMesh(core_axis_name="c", subcore_axis_name="s"),
    scratch_types=[
        pltpu.VMEM((IDX_PER_W,), jnp.int32),
        pltpu.VMEM((S, VOCAB), jnp.float32),
        pltpu.VMEM((S, 128), jnp.float32),
        pltpu.SemaphoreType.DMA,
        pltpu.SemaphoreType.DMA,
        pltpu.SemaphoreType.DMA,
    ],
    compiler_params=pltpu.CompilerParams(use_tc_tiling_on_sc=True),
)(_gather_body)


def kernel(x, embed_table, head_weight):
    table = _compute_table(embed_table, head_weight)
    tt = table.reshape(VOCAB, NBLK, 128).transpose(1, 0, 2).reshape(
        NBLK * VOCAB, 128)
    x_pad = jnp.pad(x, ((0, 0), (0, SPAD - S)))
    x_w = x_pad.reshape(NW, 1, BPW * SPAD)
    blk = (jnp.arange(NBLK, dtype=jnp.int32) * VOCAB).reshape(1, NBLK, 1)
    idx_all = (x_w + blk).reshape(NW * IDX_PER_W)
    return _gather_call(tt, idx_all)


# speedup vs baseline: 4.9476x; 3.2881x over previous
import functools

import jax
import jax.numpy as jnp
from jax import lax
from jax.experimental import pallas as pl
from jax.experimental.pallas import tpu as pltpu
from jax.experimental.pallas import tpu_sc as plsc

VOCAB = 1000
EMBED = 64
B = 1024
S = 50

NC = 2
NS = 16
NW = NC * NS
NTOK = B * S
RPW = NTOK // NW


def _e_gather_body(tbl_hbm, idx_hbm, out_hbm, idx_v, rows_v, gsem):
    wid = lax.axis_index("s") * NC + lax.axis_index("c")
    base = wid * RPW
    pltpu.sync_copy(idx_hbm.at[pl.ds(base, RPW)], idx_v)
    pltpu.async_copy(tbl_hbm.at[idx_v], rows_v, gsem).wait()
    pltpu.sync_copy(rows_v, out_hbm.at[pl.ds(base, RPW)])


_e_gather = functools.partial(
    pl.kernel,
    out_type=jax.ShapeDtypeStruct((NTOK, EMBED), jnp.float32),
    mesh=plsc.VectorSubcoreMesh(core_axis_name="c", subcore_axis_name="s"),
    scratch_types=[
        pltpu.VMEM((RPW,), jnp.int32),
        pltpu.VMEM((RPW, EMBED), jnp.float32),
        pltpu.SemaphoreType.DMA,
    ],
    compiler_params=pltpu.CompilerParams(use_tc_tiling_on_sc=False),
)(_e_gather_body)


def _proj_body(w_ref, e_ref, o_ref):
    o_ref[0] = lax.dot_general(
        w_ref[...], e_ref[0],
        dimension_numbers=(((1,), (1,)), ((), ())),
        preferred_element_type=jnp.float32,
    )


def _project(head_weight, e_sb):
    return pl.pallas_call(
        _proj_body,
        grid=(S,),
        in_specs=[
            pl.BlockSpec((VOCAB, EMBED), lambda s: (0, 0)),
            pl.BlockSpec((1, B, EMBED), lambda s: (s, 0, 0)),
        ],
        out_specs=pl.BlockSpec((1, VOCAB, B), lambda s: (s, 0, 0)),
        out_shape=jax.ShapeDtypeStruct((S, VOCAB, B), jnp.float32),
    )(head_weight, e_sb)


def kernel(x, embed_table, head_weight):
    xt = x.T.reshape(NTOK)
    e_flat = _e_gather(embed_table, xt)
    e_sb = e_flat.reshape(S, B, EMBED)
    out_svb = _project(head_weight, e_sb)
    return out_svb.transpose(2, 0, 1)


# speedup vs baseline: 5.1370x; 1.0383x over previous
import functools

import jax
import jax.numpy as jnp
from jax import lax
from jax.experimental import pallas as pl
from jax.experimental.pallas import tpu as pltpu
from jax.experimental.pallas import tpu_sc as plsc

VOCAB = 1000
EMBED = 64
B = 1024
S = 50

NC = 2
NS = 16
NW = NC * NS
NTOK = B * S
RPW = NTOK // NW


EPAD = 128
CHUNK = 400
NCHUNK = RPW // CHUNK


def _e_gather_body(tbl_hbm, idx_hbm, out_hbm, idx_v, rows0, rows1,
                   gsem0, gsem1, wsem0, wsem1):
    wid = lax.axis_index("s") * NC + lax.axis_index("c")
    base = wid * RPW
    pltpu.sync_copy(idx_hbm.at[pl.ds(base, RPW)], idx_v)

    rows = (rows0, rows1)
    gsem = (gsem0, gsem1)
    wsem = (wsem0, wsem1)

    def start_gather(c):
        return pltpu.async_copy(
            tbl_hbm.at[idx_v.at[pl.ds(c * CHUNK, CHUNK)]],
            rows[c % 2], gsem[c % 2])

    def start_write(c):
        return pltpu.async_copy(
            rows[c % 2], out_hbm.at[pl.ds(base + c * CHUNK, CHUNK)],
            wsem[c % 2])

    gathers = [None] * NCHUNK
    writes = [None] * NCHUNK
    gathers[0] = start_gather(0)
    for c in range(NCHUNK):
        gathers[c].wait()
        if c + 1 < NCHUNK:
            if c >= 1:
                writes[c - 1].wait()
            gathers[c + 1] = start_gather(c + 1)
        writes[c] = start_write(c)
    writes[NCHUNK - 1].wait()


_e_gather = functools.partial(
    pl.kernel,
    out_type=jax.ShapeDtypeStruct((NTOK, EPAD), jnp.float32),
    mesh=plsc.VectorSubcoreMesh(core_axis_name="c", subcore_axis_name="s"),
    scratch_types=[
        pltpu.VMEM((RPW,), jnp.int32),
        pltpu.VMEM((CHUNK, EPAD), jnp.float32),
        pltpu.VMEM((CHUNK, EPAD), jnp.float32),
        pltpu.SemaphoreType.DMA,
        pltpu.SemaphoreType.DMA,
        pltpu.SemaphoreType.DMA,
        pltpu.SemaphoreType.DMA,
    ],
    compiler_params=pltpu.CompilerParams(use_tc_tiling_on_sc=True),
)(_e_gather_body)


def _proj_body(w_ref, e_ref, o_ref):
    o_ref[0] = lax.dot_general(
        w_ref[...], e_ref[0, :, :EMBED],
        dimension_numbers=(((1,), (1,)), ((), ())),
        preferred_element_type=jnp.float32,
    )


def _project(head_weight, e_sb):
    return pl.pallas_call(
        _proj_body,
        grid=(S,),
        in_specs=[
            pl.BlockSpec((VOCAB, EMBED), lambda s: (0, 0)),
            pl.BlockSpec((1, B, EPAD), lambda s: (s, 0, 0)),
        ],
        out_specs=pl.BlockSpec((1, VOCAB, B), lambda s: (s, 0, 0)),
        out_shape=jax.ShapeDtypeStruct((S, VOCAB, B), jnp.float32),
    )(head_weight, e_sb)


def kernel(x, embed_table, head_weight):
    xt = x.T.reshape(NTOK)
    tbl_pad = jnp.zeros((VOCAB, EPAD), jnp.float32).at[:, :EMBED].set(
        embed_table)
    e_flat = _e_gather(tbl_pad, xt)
    e_sb = e_flat.reshape(S, B, EPAD)
    out_svb = _project(head_weight, e_sb)
    return out_svb.transpose(2, 0, 1)
